# Initial kernel scaffold; baseline (speedup 1.0000x reference)
import jax
import jax.numpy as jnp
from jax.experimental import pallas as pl


def kernel(x, xe, ye, log_temp, qindex):
    raise NotImplementedError("write your pallas kernel here")


# speedup vs baseline: 1.5946x; 1.5946x over previous
import functools

import jax
import jax.numpy as jnp
import numpy as np
from jax import lax
from jax.experimental import pallas as pl
from jax.experimental.pallas import tpu as pltpu
from jax.experimental.pallas import tpu_sc as plsc

K_NEIGH = 7
O_CAND = 50
N_PIX = 16384
M = 1024
N = 50000
FE = 256
F = 64
NPAD = 50048
OPAD = 64
ZW = K_NEIGH * F

NC, NS, LANES = 2, 16, 16
NW = NC * NS
QPW = M // NW

NBLK = 2944
NSTEPS = NPAD // NBLK


def _dist_body(ye_ref, xe_ref, d_ref):
    j = pl.program_id(0)
    ye = ye_ref[...]
    xe = xe_ref[...]
    q2 = jnp.sum(ye * ye, axis=1, keepdims=True)
    d2 = jnp.sum(xe * xe, axis=1)[None, :]
    dot = lax.dot_general(ye, xe, (((1,), (1,)), ((), ())),
                          preferred_element_type=jnp.float32)
    dist = q2 + d2 - 2.0 * dot
    col = j * NBLK + lax.broadcasted_iota(jnp.int32, (1, NBLK), 1)
    d_ref[...] = jnp.where(col >= N, jnp.float32(1e30), dist)


def _dist(ye, xep):
    return pl.pallas_call(
        _dist_body,
        grid=(NSTEPS,),
        in_specs=[
            pl.BlockSpec((M, FE), lambda j: (0, 0)),
            pl.BlockSpec((NBLK, FE), lambda j: (j, 0)),
        ],
        out_specs=pl.BlockSpec((M, NBLK), lambda j: (0, j)),
        out_shape=jax.ShapeDtypeStruct((M, NPAD), jnp.float32),
    )(ye, xep)


BUFSZ = 288
PRUNE_AT = 256
NCHB = BUFSZ // LANES
NCHROW = NPAD // LANES

_TOPBIT = np.uint32(0x80000000)


def _f32_key(v):
    b = plsc.bitcast(v, jnp.uint32)
    return jnp.where(b >= _TOPBIT, ~b, b | _TOPBIT)


def _key_val(k):
    b = jnp.where(k >= _TOPBIT, k ^ _TOPBIT, ~k)
    return plsc.bitcast(b, jnp.float32)


def _iota16():
    return lax.iota(jnp.int32, LANES)


def _popcnt(m):
    return jnp.sum(m.astype(jnp.int32))


def _prune(bufval, bufidx, keybuf, c, k):
    nch = NCHB

    def key_body(j, _):
        v = bufval[pl.ds(j * LANES, LANES)]
        keybuf[pl.ds(j * LANES, LANES)] = _f32_key(v)
        return 0
    lax.fori_loop(0, nch, key_body, 0)

    lanes = _iota16()

    def cnt_le(mid):
        def body(j, acc):
            key = keybuf[pl.ds(j * LANES, LANES)]
            lanem = (lanes + j * LANES) < c
            le = (key <= mid) & lanem
            return acc + _popcnt(le)
        return lax.fori_loop(0, nch, body, jnp.int32(0))

    def bis_body(_, lohi):
        lo, hi = lohi
        mid = lo + ((hi - lo) >> 1)
        n_le = cnt_le(mid)
        lo2 = jnp.where(n_le >= k, lo, mid + 1)
        hi2 = jnp.where(n_le >= k, mid, hi)
        return (lo2, hi2)

    zero16 = jnp.zeros((LANES,), jnp.uint32)
    lo0 = zero16
    hi0 = zero16 + jnp.uint32(0xFFFFFFFF)
    lo, _ = lax.fori_loop(0, 32, bis_body, (lo0, hi0))
    tkey = lo

    def cl_body(j, acc):
        key = keybuf[pl.ds(j * LANES, LANES)]
        lanem = (lanes + j * LANES) < c
        lt = (key < tkey) & lanem
        return acc + _popcnt(lt)
    cl = lax.fori_loop(0, nch, cl_body, jnp.int32(0))
    need_eq = k - cl

    def comp_body(j, carry):
        c2, taken = carry
        base = j * LANES
        key = keybuf[pl.ds(base, LANES)]
        val = bufval[pl.ds(base, LANES)]
        idx = bufidx[pl.ds(base, LANES)]
        lanem = (lanes + base) < c
        mlt = (key < tkey) & lanem
        meq = (key == tkey) & lanem
        rank = plsc.cumsum(meq.astype(jnp.int32))
        take_eq = meq & ((taken + rank) <= need_eq)
        take = mlt | take_eq
        plsc.store_compressed(bufval.at[pl.ds(c2, LANES)], val, mask=take)
        plsc.store_compressed(bufidx.at[pl.ds(c2, LANES)], idx, mask=take)
        return (c2 + _popcnt(take), taken + _popcnt(take_eq))
    lax.fori_loop(0, nch, comp_body, (jnp.int32(0), jnp.int32(0)))

    return jnp.zeros((LANES,), jnp.float32) + _key_val(tkey)


def _scan_row(rowbuf, bufval, bufidx, keybuf):
    lanes = _iota16()
    t0 = jnp.zeros((LANES,), jnp.float32) + jnp.float32(jnp.inf)

    def chunk_body(i, carry):
        t, c = carry
        v = rowbuf[pl.ds(i * LANES, LANES)]
        m = v < t
        n = _popcnt(m)

        def append(args):
            t_, c_ = args
            idxv = lanes + i * LANES
            plsc.store_compressed(bufval.at[pl.ds(c_, LANES)], v, mask=m)
            plsc.store_compressed(bufidx.at[pl.ds(c_, LANES)], idxv, mask=m)
            c2 = c_ + n

            def do_prune(args2):
                _, _ = args2
                tn = _prune(bufval, bufidx, keybuf, c2, O_CAND)
                return (tn, jnp.int32(O_CAND))

            return lax.cond(c2 >= PRUNE_AT, do_prune, lambda a: (t_, c2),
                            (t_, c2))

        return lax.cond(n > 0, append, lambda a: a, (t, c))

    t, c = lax.fori_loop(0, NCHROW, chunk_body, (t0, jnp.int32(0)))
    _prune(bufval, bufidx, keybuf, c, O_CAND)


def _topk_body(dist_hbm, x_hbm, vals_hbm, xg_hbm,
               rowbuf, bufval, bufidx, keybuf, outv, outi, xgbuf,
               sem0, sem1, gsem):
    c = lax.axis_index("c")
    s = lax.axis_index("s")
    wid = s * NC + c
    qbase = wid * QPW
    lanes = _iota16()

    def finish_query(q):
        inf16 = jnp.zeros((LANES,), jnp.float32) + np.float32(np.inf)
        for j in range(OPAD // LANES):
            outv[pl.ds(j * LANES, LANES)] = (jnp.zeros((LANES,), jnp.float32)
                                             + np.float32(1e30))
            outi[pl.ds(j * LANES, LANES)] = jnp.zeros((LANES,), jnp.int32)
        ws = []
        idxs = []
        for j in range(4):
            sel = (lanes + j * LANES) < O_CAND
            ws.append(jnp.where(sel, bufval[pl.ds(j * LANES, LANES)], inf16))
            idxs.append(bufidx[pl.ds(j * LANES, LANES)])

        def sel_body(pos, carry):
            w0, w1, w2, w3 = carry
            mn = jnp.min(jnp.minimum(jnp.minimum(w0, w1),
                                     jnp.minimum(w2, w3)))
            out = []
            prev_scalar = jnp.int32(0)
            for j, wj in enumerate((w0, w1, w2, w3)):
                eq = wj == mn
                csum = plsc.cumsum(eq.astype(jnp.int32))
                take = eq & (csum == 1) & (prev_scalar == 0)
                plsc.store_compressed(outv.at[pl.ds(pos, LANES)], wj,
                                      mask=take)
                plsc.store_compressed(outi.at[pl.ds(pos, LANES)], idxs[j],
                                      mask=take)
                prev_scalar = prev_scalar + _popcnt(eq)
                out.append(jnp.where(take, inf16, wj))
            return tuple(out)

        lax.fori_loop(0, O_CAND, sel_body, tuple(ws))
        pltpu.sync_copy(outv.at[pl.ds(0, OPAD)], vals_hbm.at[q])
        pltpu.async_copy(x_hbm.at[outi.at[pl.ds(0, OPAD)]], xgbuf,
                         gsem).wait()
        pltpu.sync_copy(xgbuf, xg_hbm.at[q])

    q0 = qbase
    cp0 = pltpu.async_copy(dist_hbm.at[q0], rowbuf.at[0], sem0)

    def pair_body(p, _):
        qe = qbase + 2 * p
        pltpu.make_async_copy(dist_hbm.at[qe], rowbuf.at[0], sem0).wait()
        pltpu.async_copy(dist_hbm.at[qe + 1], rowbuf.at[1], sem1)
        _scan_row(rowbuf.at[0], bufval, bufidx, keybuf)
        finish_query(qe)
        pltpu.make_async_copy(dist_hbm.at[qe + 1], rowbuf.at[1], sem1).wait()

        @pl.when(p + 1 < QPW // 2)
        def _():
            pltpu.async_copy(dist_hbm.at[qe + 2], rowbuf.at[0], sem0)

        _scan_row(rowbuf.at[1], bufval, bufidx, keybuf)
        finish_query(qe + 1)
        return 0

    lax.fori_loop(0, QPW // 2, pair_body, 0)
    del cp0


def _topk_gather(dist, x):
    mesh = plsc.VectorSubcoreMesh(core_axis_name="c", subcore_axis_name="s",
                                  num_cores=NC, num_subcores=NS)
    f = functools.partial(
        pl.kernel,
        out_type=[
            jax.ShapeDtypeStruct((M, OPAD), jnp.float32),
            jax.ShapeDtypeStruct((M, OPAD, F), jnp.float32),
        ],
        mesh=mesh,
        compiler_params=pltpu.CompilerParams(use_tc_tiling_on_sc=False,
                                             needs_layout_passes=False),
        scratch_types=[
            pltpu.VMEM((2, NPAD), jnp.float32),
            pltpu.VMEM((BUFSZ,), jnp.float32),
            pltpu.VMEM((BUFSZ,), jnp.int32),
            pltpu.VMEM((BUFSZ,), jnp.uint32),
            pltpu.VMEM((OPAD + LANES,), jnp.float32),
            pltpu.VMEM((OPAD + LANES,), jnp.int32),
            pltpu.VMEM((OPAD, F), jnp.float32),
            pltpu.SemaphoreType.DMA,
            pltpu.SemaphoreType.DMA,
            pltpu.SemaphoreType.DMA,
        ],
    )(_topk_body)
    return f(dist, x)


MB = 128


def _lane_sum64(x):
    s = x
    w = 32
    while w >= 1:
        s = s[:, :w] + s[:, w:2 * w]
        w //= 2
    return s


def _cascade_body(vals_ref, lt_ref, xg_ref, z_ref):
    vals = vals_ref[...]
    lt = _lane_sum64(lt_ref[...]) / 64.0
    temp = jnp.exp(lt)
    cur = (-vals) / temp
    xg = xg_ref[...]
    for k in range(K_NEIGH):
        mx = jnp.max(cur, axis=1, keepdims=True)
        e = jnp.exp(cur - mx)
        w = e / _lane_sum64(e)
        zk = jnp.sum(w[:, :, None] * xg, axis=1)
        z_ref[k, :, :] = zk
        cur = cur + jnp.log(jnp.clip(1.0 - w, 1e-10, 1.0))


def _cascade(vals, log_temp, xg):
    return pl.pallas_call(
        _cascade_body,
        grid=(M // MB,),
        in_specs=[
            pl.BlockSpec((MB, OPAD), lambda i: (i, 0)),
            pl.BlockSpec((MB, F), lambda i: (i, 0)),
            pl.BlockSpec((MB, OPAD, F), lambda i: (i, 0, 0)),
        ],
        out_specs=pl.BlockSpec((K_NEIGH, MB, F), lambda i: (0, i, 0)),
        out_shape=jax.ShapeDtypeStruct((K_NEIGH, M, F), jnp.float32),
    )(vals, log_temp, xg)


NPASS = 4
RSLAB = N_PIX // (2 * NPASS)
TSLAB = RSLAB // NS
QPT = M // NS
SPAD = RSLAB + 8


def _fold_body(z_hbm, qidx_hbm, acc_hbm, spacc, zrows, qv, relv, junk):
    c = lax.axis_index("c")
    s = lax.axis_index("s")

    def one_pass(p, _):
        row_base = (c * NPASS + p) * RSLAB

        def zb(r, _2):
            def zc(j, _3):
                zrows[r, pl.ds(j * LANES, LANES)] = jnp.zeros((LANES,),
                                                              jnp.float32)
                return 0
            return lax.fori_loop(0, ZW // LANES, zc, 0)
        lax.fori_loop(0, QPT, zb, 0)

        def zslab(k, _2):
            pltpu.sync_copy(zrows, spacc.at[pl.ds(s * TSLAB + k * 64, 64)])
            return 0
        lax.fori_loop(0, TSLAB // 64, zslab, 0)

        @pl.when(s == 0)
        def _():
            pltpu.sync_copy(zrows.at[pl.ds(0, 8)], spacc.at[pl.ds(RSLAB, 8)])

        plsc.subcore_barrier()

        pltpu.sync_copy(qidx_hbm.at[pl.ds(s * QPT, QPT)], qv)

        def rel_body(j, _):
            iv = qv[pl.ds(j * LANES, LANES)]
            rel = iv - row_base
            oob = (rel < 0) | (rel >= RSLAB)
            relv[pl.ds(j * LANES, LANES)] = jnp.where(oob, RSLAB, rel)
            return 0
        lax.fori_loop(0, QPT // LANES, rel_body, 0)

        pltpu.sync_copy(z_hbm.at[pl.ds(s * QPT, QPT)], zrows)
        pltpu.sync_copy(zrows, spacc.at[relv], add=True)
        plsc.subcore_barrier()

        pltpu.sync_copy(spacc.at[pl.ds(s * TSLAB, TSLAB)],
                        acc_hbm.at[pl.ds(row_base + s * TSLAB, TSLAB)])
        plsc.subcore_barrier()
        return 0

    lax.fori_loop(0, NPASS, one_pass, 0)
    del junk


def _fold(zflat, qindex):
    mesh = plsc.VectorSubcoreMesh(core_axis_name="c", subcore_axis_name="s",
                                  num_cores=NC, num_subcores=NS)
    f = functools.partial(
        pl.kernel,
        out_type=[jax.ShapeDtypeStruct((N_PIX, ZW), jnp.float32)],
        mesh=mesh,
        compiler_params=pltpu.CompilerParams(use_tc_tiling_on_sc=False,
                                             needs_layout_passes=False),
        scratch_types=[
            pltpu.VMEM_SHARED((SPAD, ZW), jnp.float32),
            pltpu.VMEM((QPT, ZW), jnp.float32),
            pltpu.VMEM((QPT,), jnp.int32),
            pltpu.VMEM((QPT,), jnp.int32),
            pltpu.SemaphoreType.DMA,
        ],
    )(_fold_body)
    return f(zflat, qindex)[0]


RB = 512


def _norm_body(acc_ref, qidx_ref, out_ref):
    i = pl.program_id(0)
    rows = i * RB + lax.broadcasted_iota(jnp.int32, (RB, 1), 0)
    q = qidx_ref[...]
    cnt = jnp.sum((q == rows).astype(jnp.float32), axis=1, keepdims=True)
    out_ref[...] = acc_ref[...] / jnp.maximum(cnt, 1.0)


def _normalize(acc, qidx2d):
    return pl.pallas_call(
        _norm_body,
        grid=(N_PIX // RB,),
        in_specs=[
            pl.BlockSpec((RB, ZW), lambda i: (i, 0)),
            pl.BlockSpec((1, M), lambda i: (0, 0)),
        ],
        out_specs=pl.BlockSpec((RB, ZW), lambda i: (i, 0)),
        out_shape=jax.ShapeDtypeStruct((N_PIX, ZW), jnp.float32),
    )(acc, qidx2d)


def kernel(x, xe, ye, log_temp, qindex):
    xep = jnp.pad(xe, ((0, NPAD - N), (0, 0)))
    dist = _dist(ye, xep)
    vals, xg = _topk_gather(dist, x)
    z3 = _cascade(vals, log_temp, xg)
    zflat = z3.transpose(1, 0, 2).reshape(M, ZW)
    acc = _fold(zflat, qindex.astype(jnp.int32))
    return _normalize(acc, qindex.astype(jnp.int32).reshape(1, M))


# speedup vs baseline: 3.1522x; 1.9767x over previous
import functools

import jax
import jax.numpy as jnp
import numpy as np
from jax import lax
from jax.experimental import pallas as pl
from jax.experimental.pallas import tpu as pltpu
from jax.experimental.pallas import tpu_sc as plsc

K_NEIGH = 7
O_CAND = 50
N_PIX = 16384
M = 1024
N = 50000
FE = 256
F = 64
NPAD = 50048
OPAD = 64
ZW = K_NEIGH * F

NC, NS, LANES = 2, 16, 16
NW = NC * NS
QPW = M // NW

NBLK = 2944
NSTEPS = NPAD // NBLK


def _dist_body(ye_ref, xe_ref, d_ref):
    j = pl.program_id(0)
    ye = ye_ref[...]
    xe = xe_ref[...]
    q2 = jnp.sum(ye * ye, axis=1, keepdims=True)
    d2 = jnp.sum(xe * xe, axis=1)[None, :]
    dot = lax.dot_general(ye, xe, (((1,), (1,)), ((), ())),
                          preferred_element_type=jnp.float32)
    dist = q2 + d2 - 2.0 * dot
    col = j * NBLK + lax.broadcasted_iota(jnp.int32, (1, NBLK), 1)
    d_ref[...] = jnp.where(col >= N, jnp.float32(1e30), dist)


def _dist(ye, xep):
    return pl.pallas_call(
        _dist_body,
        grid=(NSTEPS,),
        in_specs=[
            pl.BlockSpec((M, FE), lambda j: (0, 0)),
            pl.BlockSpec((NBLK, FE), lambda j: (j, 0)),
        ],
        out_specs=pl.BlockSpec((M, NBLK), lambda j: (0, j)),
        out_shape=jax.ShapeDtypeStruct((M, NPAD), jnp.float32),
    )(ye, xep)


BUFSZ = 528
PRUNE_AT = 384
NCHB = BUFSZ // LANES
NCHROW = NPAD // LANES
USCAN = 8
_MANT = np.int32(0x7FFFFFFF)


def _f32_key(v):
    b = plsc.bitcast(v, jnp.int32)
    return jnp.where(b >= 0, b, b ^ _MANT)


def _key_val(k):
    b = jnp.where(k >= 0, k, k ^ _MANT)
    return plsc.bitcast(b, jnp.float32)


def _iota16():
    return lax.iota(jnp.int32, LANES)


def _popcnt(m):
    return jnp.sum(m.astype(jnp.int32))


def _prune(bufval, bufidx, keybuf, c, k):
    nch = NCHB

    def key_body(j, _):
        v = bufval[pl.ds(j * LANES, LANES)]
        keybuf[pl.ds(j * LANES, LANES)] = _f32_key(v)
        return 0
    lax.fori_loop(0, nch, key_body, 0)

    lanes = _iota16()

    zero16 = jnp.zeros((LANES,), jnp.int32)

    def cnt_le(mid):
        def body(j, acc):
            key = keybuf[pl.ds(j * LANES, LANES)]
            lanem = (lanes + j * LANES) < c
            le = (key <= mid) & lanem
            return acc + le.astype(jnp.int32)
        accv = lax.fori_loop(0, nch, body, zero16)
        return jnp.sum(accv)

    def bis_body(_, lohi):
        lo, hi = lohi
        mid = (lo >> 1) + (hi >> 1) + (lo & hi & 1)
        n_le = cnt_le(mid)
        lo2 = jnp.where(n_le >= k, lo, mid + 1)
        hi2 = jnp.where(n_le >= k, mid, hi)
        return (lo2, hi2)

    lo0 = zero16 + np.int32(-2147483648)
    hi0 = zero16 + np.int32(2147483647)
    lo, _ = lax.fori_loop(0, 32, bis_body, (lo0, hi0))
    tkey = lo

    def cl_body(j, acc):
        key = keybuf[pl.ds(j * LANES, LANES)]
        lanem = (lanes + j * LANES) < c
        lt = (key < tkey) & lanem
        return acc + lt.astype(jnp.int32)
    cl = jnp.sum(lax.fori_loop(0, nch, cl_body, zero16))
    need_eq = k - cl

    def comp_body(j, carry):
        c2, taken = carry
        base = j * LANES
        key = keybuf[pl.ds(base, LANES)]
        val = bufval[pl.ds(base, LANES)]
        idx = bufidx[pl.ds(base, LANES)]
        lanem = (lanes + base) < c
        mlt = (key < tkey) & lanem
        meq = (key == tkey) & lanem
        rank = plsc.cumsum(meq.astype(jnp.int32))
        take_eq = meq & ((taken + rank) <= need_eq)
        take = mlt | take_eq
        plsc.store_compressed(bufval.at[pl.ds(c2, LANES)], val, mask=take)
        plsc.store_compressed(bufidx.at[pl.ds(c2, LANES)], idx, mask=take)
        return (c2 + _popcnt(take), taken + _popcnt(take_eq))
    lax.fori_loop(0, nch, comp_body, (jnp.int32(0), jnp.int32(0)))

    return jnp.zeros((LANES,), jnp.float32) + _key_val(tkey)


def _scan_row(rowbuf, bufval, bufidx, keybuf):
    lanes = _iota16()
    t0 = jnp.zeros((LANES,), jnp.float32) + np.float32(np.inf)

    def blk_body(ib, carry):
        t, c = carry
        base = ib * USCAN * LANES
        vs = [rowbuf[pl.ds(base + u * LANES, LANES)] for u in range(USCAN)]
        anym = vs[0] < t
        for u in range(1, USCAN):
            anym = anym | (vs[u] < t)
        n_any = _popcnt(anym)

        def append(args):
            t_, c_ = args
            for u in range(USCAN):
                mu = vs[u] < t_
                idxv = lanes + (base + u * LANES)
                plsc.store_compressed(bufval.at[pl.ds(c_, LANES)], vs[u],
                                      mask=mu)
                plsc.store_compressed(bufidx.at[pl.ds(c_, LANES)], idxv,
                                      mask=mu)
                c_ = c_ + _popcnt(mu)

            def do_prune(args2):
                _, _ = args2
                tn = _prune(bufval, bufidx, keybuf, c_, O_CAND)
                return (tn, jnp.int32(O_CAND))

            return lax.cond(c_ >= PRUNE_AT, do_prune, lambda a: (t_, c_),
                            (t_, c_))

        return lax.cond(n_any > 0, append, lambda a: a, (t, c))

    t, c = lax.fori_loop(0, NCHROW // USCAN, blk_body, (t0, jnp.int32(0)))
    _prune(bufval, bufidx, keybuf, c, O_CAND)


def _topk_body(dist_hbm, x_hbm, vals_hbm, xg_hbm,
               rowbuf, bufval, bufidx, keybuf, outv, outi, xgbuf,
               sem0, sem1, gsem):
    c = lax.axis_index("c")
    s = lax.axis_index("s")
    wid = s * NC + c
    qbase = wid * QPW
    lanes = _iota16()

    def finish_query(q):
        ks = []
        xs = []
        for j in range(4):
            sel = (lanes + j * LANES) < O_CAND
            v = jnp.where(sel, bufval[pl.ds(j * LANES, LANES)],
                          np.float32(1e30))
            ii = jnp.where(sel, bufidx[pl.ds(j * LANES, LANES)], 0)
            k, x = plsc.sort_key_val(_f32_key(v), ii)
            ks.append(k)
            xs.append(x)

        def cmpx(ka, xa, kb, xb):
            sel = ka <= kb
            return (jnp.where(sel, ka, kb), jnp.where(sel, xa, xb),
                    jnp.where(sel, kb, ka), jnp.where(sel, xb, xa))

        def rev(v):
            return lax.rev(v, (0,))

        def merge16(ka, xa, kb, xb):
            lk, lx, hk, hx = cmpx(ka, xa, rev(kb), rev(xb))
            l = plsc.sort_key_val(lk, lx)
            h = plsc.sort_key_val(hk, hx)
            return l[0], l[1], h[0], h[1]

        a0, ax0, a1, ax1 = merge16(ks[0], xs[0], ks[1], xs[1])
        b0, bx0, b1, bx1 = merge16(ks[2], xs[2], ks[3], xs[3])
        l0k, l0x, h0k, h0x = cmpx(a0, ax0, rev(b1), rev(bx1))
        l1k, l1x, h1k, h1x = cmpx(a1, ax1, rev(b0), rev(bx0))
        p0k, p0x, p1k, p1x = cmpx(l0k, l0x, l1k, l1x)
        q0k, q0x, q1k, q1x = cmpx(h0k, h0x, h1k, h1x)
        s0 = plsc.sort_key_val(p0k, p0x)
        s1 = plsc.sort_key_val(p1k, p1x)
        s2 = plsc.sort_key_val(q0k, q0x)
        s3 = plsc.sort_key_val(q1k, q1x)
        for j, (kk, xx) in enumerate((s0, s1, s2, s3)):
            outv[pl.ds(j * LANES, LANES)] = _key_val(kk)
            outi[pl.ds(j * LANES, LANES)] = xx
        pltpu.sync_copy(outv.at[pl.ds(0, OPAD)], vals_hbm.at[q])
        pltpu.async_copy(x_hbm.at[outi.at[pl.ds(0, OPAD)]], xgbuf,
                         gsem).wait()
        pltpu.sync_copy(xgbuf, xg_hbm.at[q])

    q0 = qbase
    cp0 = pltpu.async_copy(dist_hbm.at[q0], rowbuf.at[0], sem0)

    def pair_body(p, _):
        qe = qbase + 2 * p
        pltpu.make_async_copy(dist_hbm.at[qe], rowbuf.at[0], sem0).wait()
        pltpu.async_copy(dist_hbm.at[qe + 1], rowbuf.at[1], sem1)
        _scan_row(rowbuf.at[0], bufval, bufidx, keybuf)
        finish_query(qe)
        pltpu.make_async_copy(dist_hbm.at[qe + 1], rowbuf.at[1], sem1).wait()

        @pl.when(p + 1 < QPW // 2)
        def _():
            pltpu.async_copy(dist_hbm.at[qe + 2], rowbuf.at[0], sem0)

        _scan_row(rowbuf.at[1], bufval, bufidx, keybuf)
        finish_query(qe + 1)
        return 0

    lax.fori_loop(0, QPW // 2, pair_body, 0)
    del cp0


def _topk_gather(dist, x):
    mesh = plsc.VectorSubcoreMesh(core_axis_name="c", subcore_axis_name="s",
                                  num_cores=NC, num_subcores=NS)
    f = functools.partial(
        pl.kernel,
        out_type=[
            jax.ShapeDtypeStruct((M, OPAD), jnp.float32),
            jax.ShapeDtypeStruct((M, OPAD, F), jnp.float32),
        ],
        mesh=mesh,
        compiler_params=pltpu.CompilerParams(use_tc_tiling_on_sc=False,
                                             needs_layout_passes=False),
        scratch_types=[
            pltpu.VMEM((2, NPAD), jnp.float32),
            pltpu.VMEM((BUFSZ,), jnp.float32),
            pltpu.VMEM((BUFSZ,), jnp.int32),
            pltpu.VMEM((BUFSZ,), jnp.int32),
            pltpu.VMEM((OPAD + LANES,), jnp.float32),
            pltpu.VMEM((OPAD + LANES,), jnp.int32),
            pltpu.VMEM((OPAD, F), jnp.float32),
            pltpu.SemaphoreType.DMA,
            pltpu.SemaphoreType.DMA,
            pltpu.SemaphoreType.DMA,
        ],
    )(_topk_body)
    return f(dist, x)


MB = 128


def _lane_sum64(x):
    s = x
    w = 32
    while w >= 1:
        s = s[:, :w] + s[:, w:2 * w]
        w //= 2
    return s


def _cascade_body(vals_ref, lt_ref, xg_ref, z_ref):
    vals = vals_ref[...]
    lt = _lane_sum64(lt_ref[...]) / 64.0
    temp = jnp.exp(lt)
    cur = (-vals) / temp
    xg = xg_ref[...]
    for k in range(K_NEIGH):
        mx = jnp.max(cur, axis=1, keepdims=True)
        e = jnp.exp(cur - mx)
        w = e / _lane_sum64(e)
        zk = jnp.sum(w[:, :, None] * xg, axis=1)
        z_ref[k, :, :] = zk
        cur = cur + jnp.log(jnp.clip(1.0 - w, 1e-10, 1.0))


def _cascade(vals, log_temp, xg):
    return pl.pallas_call(
        _cascade_body,
        grid=(M // MB,),
        in_specs=[
            pl.BlockSpec((MB, OPAD), lambda i: (i, 0)),
            pl.BlockSpec((MB, F), lambda i: (i, 0)),
            pl.BlockSpec((MB, OPAD, F), lambda i: (i, 0, 0)),
        ],
        out_specs=pl.BlockSpec((K_NEIGH, MB, F), lambda i: (0, i, 0)),
        out_shape=jax.ShapeDtypeStruct((K_NEIGH, M, F), jnp.float32),
    )(vals, log_temp, xg)


NPASS = 4
RSLAB = N_PIX // (2 * NPASS)
TSLAB = RSLAB // NS
QPT = M // NS
SPAD = RSLAB + 8


def _fold_body(z_hbm, qidx_hbm, acc_hbm, spacc, zrows, qv, relv, junk):
    c = lax.axis_index("c")
    s = lax.axis_index("s")

    def one_pass(p, _):
        row_base = (c * NPASS + p) * RSLAB

        def zb(r, _2):
            def zc(j, _3):
                zrows[r, pl.ds(j * LANES, LANES)] = jnp.zeros((LANES,),
                                                              jnp.float32)
                return 0
            return lax.fori_loop(0, ZW // LANES, zc, 0)
        lax.fori_loop(0, QPT, zb, 0)

        def zslab(k, _2):
            pltpu.sync_copy(zrows, spacc.at[pl.ds(s * TSLAB + k * 64, 64)])
            return 0
        lax.fori_loop(0, TSLAB // 64, zslab, 0)

        @pl.when(s == 0)
        def _():
            pltpu.sync_copy(zrows.at[pl.ds(0, 8)], spacc.at[pl.ds(RSLAB, 8)])

        plsc.subcore_barrier()

        pltpu.sync_copy(qidx_hbm.at[pl.ds(s * QPT, QPT)], qv)

        def rel_body(j, _):
            iv = qv[pl.ds(j * LANES, LANES)]
            rel = iv - row_base
            oob = (rel < 0) | (rel >= RSLAB)
            relv[pl.ds(j * LANES, LANES)] = jnp.where(oob, RSLAB, rel)
            return 0
        lax.fori_loop(0, QPT // LANES, rel_body, 0)

        pltpu.sync_copy(z_hbm.at[pl.ds(s * QPT, QPT)], zrows)
        pltpu.sync_copy(zrows, spacc.at[relv], add=True)
        plsc.subcore_barrier()

        pltpu.sync_copy(spacc.at[pl.ds(s * TSLAB, TSLAB)],
                        acc_hbm.at[pl.ds(row_base + s * TSLAB, TSLAB)])
        plsc.subcore_barrier()
        return 0

    lax.fori_loop(0, NPASS, one_pass, 0)
    del junk


def _fold(zflat, qindex):
    mesh = plsc.VectorSubcoreMesh(core_axis_name="c", subcore_axis_name="s",
                                  num_cores=NC, num_subcores=NS)
    f = functools.partial(
        pl.kernel,
        out_type=[jax.ShapeDtypeStruct((N_PIX, ZW), jnp.float32)],
        mesh=mesh,
        compiler_params=pltpu.CompilerParams(use_tc_tiling_on_sc=False,
                                             needs_layout_passes=False),
        scratch_types=[
            pltpu.VMEM_SHARED((SPAD, ZW), jnp.float32),
            pltpu.VMEM((QPT, ZW), jnp.float32),
            pltpu.VMEM((QPT,), jnp.int32),
            pltpu.VMEM((QPT,), jnp.int32),
            pltpu.SemaphoreType.DMA,
        ],
    )(_fold_body)
    return f(zflat, qindex)[0]


RB = 512


def _norm_body(acc_ref, qidx_ref, out_ref):
    i = pl.program_id(0)
    rows = i * RB + lax.broadcasted_iota(jnp.int32, (RB, 1), 0)
    q = qidx_ref[...]
    cnt = jnp.sum((q == rows).astype(jnp.float32), axis=1, keepdims=True)
    out_ref[...] = acc_ref[...] / jnp.maximum(cnt, 1.0)


def _normalize(acc, qidx2d):
    return pl.pallas_call(
        _norm_body,
        grid=(N_PIX // RB,),
        in_specs=[
            pl.BlockSpec((RB, ZW), lambda i: (i, 0)),
            pl.BlockSpec((1, M), lambda i: (0, 0)),
        ],
        out_specs=pl.BlockSpec((RB, ZW), lambda i: (i, 0)),
        out_shape=jax.ShapeDtypeStruct((N_PIX, ZW), jnp.float32),
    )(acc, qidx2d)


def kernel(x, xe, ye, log_temp, qindex):
    xep = jnp.pad(xe, ((0, NPAD - N), (0, 0)))
    dist = _dist(ye, xep)
    vals, xg = _topk_gather(dist, x)
    z3 = _cascade(vals, log_temp, xg)
    zflat = z3.transpose(1, 0, 2).reshape(M, ZW)
    acc = _fold(zflat, qindex.astype(jnp.int32))
    return _normalize(acc, qindex.astype(jnp.int32).reshape(1, M))
